# baseline (speedup 1.0000x reference)
import functools

import jax
import jax.numpy as jnp
from jax import lax
from jax.experimental import pallas as pl
from jax.experimental.pallas import tpu as pltpu
from jax.experimental.pallas import tpu_sc as plsc

L = 16
NC = 2
NS = 16
NW = NC * NS
GROUP = 128


def _elu_body(x_ref, w_ref, o_ref):
    x = x_ref[...] * w_ref[...]
    o_ref[...] = jnp.where(x > 0, x, jnp.exp(jnp.minimum(x, 0.0)) - 1.0)


def _add_body(p_ref, o_ref):
    o_ref[...] = p_ref[0] + p_ref[1]


def _make_sc_call(n_nodes, d, gpt0, gpt1, acc_rows, trash):
    icg = 16
    cbuf_len = (icg + 1) * GROUP + L
    wrows = -(-n_nodes // (NS * 8)) * 8
    wlast = n_nodes - (NS - 1) * wrows
    zper = acc_rows // NS
    mesh = plsc.VectorSubcoreMesh(core_axis_name="c", subcore_axis_name="s")

    @functools.partial(
        pl.kernel,
        out_type=jax.ShapeDtypeStruct((NC, n_nodes, d), jnp.float32),
        mesh=mesh,
        compiler_params=pltpu.CompilerParams(needs_layout_passes=False),
        scratch_types=[
            pltpu.VMEM((icg, GROUP), jnp.int32),
            pltpu.VMEM((icg, GROUP), jnp.int32),
            pltpu.VMEM((icg, GROUP), jnp.int32),
            pltpu.VMEM((cbuf_len,), jnp.int32),
            pltpu.VMEM((cbuf_len,), jnp.int32),
            pltpu.VMEM((2, GROUP), jnp.int32),
            pltpu.VMEM((GROUP, d), jnp.float32),
            pltpu.VMEM((GROUP, d), jnp.float32),
            pltpu.VMEM_SHARED((acc_rows, d), jnp.float32),
            pltpu.SemaphoreType.DMA,
            pltpu.SemaphoreType.DMA,
            pltpu.SemaphoreType.DMA,
            pltpu.SemaphoreType.DMA,
        ],
    )
    def sc_kernel(emb_h, src1_h, dst1_h, src2_h, dst2_h, ef_h, out_h,
                  idx_src, idx_dst, idx_e, csrc, cdst, hrow,
                  rows0, rows1, acc, gsem0, gsem1, ssem0, ssem1):
        rows_bufs = (rows0, rows1)
        gsems = (gsem0, gsem1)
        ssems = (ssem0, ssem1)
        c = lax.axis_index("c")
        s = lax.axis_index("s")

        def zrow(r, carry):
            for k in range(d // L):
                rows0[r, pl.ds(k * L, L)] = jnp.zeros((L,), jnp.float32)
            return carry
        lax.fori_loop(0, GROUP, zrow, 0)
        zbase = s * zper
        def zcp(i, carry):
            pltpu.sync_copy(rows0, acc.at[pl.ds(zbase + i * GROUP, GROUP)])
            return carry
        lax.fori_loop(0, zper // GROUP, zcp, 0)
        plsc.subcore_barrier()

        my_gpt = jnp.where(c == 0, gpt0, gpt1)
        row_base = jnp.where(c == 0, s * gpt0, NS * gpt0 + s * gpt1)

        tvec = jnp.full((L,), trash, jnp.int32) + s

        def run_compacted(off):
            for k in range(GROUP // L):
                csrc[pl.ds(off + k * L, L)] = jnp.zeros((L,), jnp.int32)
                cdst[pl.ds(off + k * L, L)] = tvec
            ng = off // GROUP + 1

            pltpu.async_copy(emb_h.at[csrc.at[pl.ds(0, GROUP)]], rows0,
                             gsem0)

            def pair2(gg, carry2):
                for b in range(2):
                    g2 = gg * 2 + b

                    @pl.when(g2 < ng)
                    def _():
                        rb, rnb = rows_bufs[b], rows_bufs[1 - b]

                        @pl.when(g2 + 1 < ng)
                        def _():
                            @pl.when(g2 >= 1)
                            def _():
                                pltpu.make_async_copy(
                                    rnb, acc.at[hrow.at[1 - b]],
                                    ssems[1 - b]).wait()
                            pltpu.async_copy(
                                emb_h.at[
                                    csrc.at[pl.ds((g2 + 1) * GROUP, GROUP)]],
                                rnb, gsems[1 - b])

                        pltpu.make_async_copy(
                            emb_h.at[csrc.at[pl.ds(0, GROUP)]], rb,
                            gsems[b]).wait()

                        for k in range(GROUP // L):
                            hrow[b, pl.ds(k * L, L)] = cdst[
                                pl.ds(g2 * GROUP + k * L, L)]

                        pltpu.async_copy(rb, acc.at[hrow.at[b]], ssems[b],
                                         add=True)
                return carry2
            lax.fori_loop(0, (icg + 2) // 2, pair2, 0)

            nm = ng % 2

            @pl.when(ng >= 2)
            def _():
                @pl.when(nm == 0)
                def _():
                    pltpu.make_async_copy(
                        rows0, acc.at[hrow.at[0]], ssem0).wait()

                @pl.when(nm == 1)
                def _():
                    pltpu.make_async_copy(
                        rows1, acc.at[hrow.at[1]], ssem1).wait()

            @pl.when(nm == 1)
            def _():
                pltpu.make_async_copy(
                    rows0, acc.at[hrow.at[0]], ssem0).wait()

            @pl.when(nm == 0)
            def _():
                pltpu.make_async_copy(
                    rows1, acc.at[hrow.at[1]], ssem1).wait()

        def ichunk1(ic, carry):
            ib = row_base + ic * icg
            pltpu.sync_copy(src1_h.at[pl.ds(ib, icg)], idx_src)
            pltpu.sync_copy(dst1_h.at[pl.ds(ib, icg)], idx_dst)
            pltpu.sync_copy(ef_h.at[pl.ds(ib, icg)], idx_e)

            pltpu.async_copy(emb_h.at[idx_src.at[0]], rows0, gsem0)

            def pair(gg, off):
                for b in range(2):
                    g = gg * 2 + b
                    rb, rnb = rows_bufs[b], rows_bufs[1 - b]

                    @pl.when(g + 1 < icg)
                    def _():
                        @pl.when(g >= 1)
                        def _():
                            pltpu.make_async_copy(
                                rnb, acc.at[idx_dst.at[0]],
                                ssems[1 - b]).wait()
                        pltpu.async_copy(
                            emb_h.at[idx_src.at[g + 1]], rnb, gsems[1 - b])

                    for j in range(GROUP // L):
                        ev = idx_e[g, pl.ds(j * L, L)]
                        m = (ev >= 0) & (ev <= 4)
                        plsc.store_compressed(
                            csrc.at[pl.ds(off, L)],
                            idx_src[g, pl.ds(j * L, L)], mask=m)
                        plsc.store_compressed(
                            cdst.at[pl.ds(off, L)],
                            idx_dst[g, pl.ds(j * L, L)], mask=m)
                        off = off + jnp.sum(m.astype(jnp.int32))

                    pltpu.make_async_copy(
                        emb_h.at[idx_src.at[g]], rb, gsems[b]).wait()
                    pltpu.async_copy(rb, acc.at[idx_dst.at[g]], ssems[b],
                                     add=True)
                return off
            off = lax.fori_loop(0, icg // 2, pair, jnp.int32(0))

            pltpu.make_async_copy(rows0, acc.at[idx_dst.at[0]],
                                  ssem0).wait()
            pltpu.make_async_copy(rows1, acc.at[idx_dst.at[0]],
                                  ssem1).wait()

            run_compacted(off)
            return carry
        lax.fori_loop(0, my_gpt // icg, ichunk1, 0)

        def ichunk2(ic, carry):
            ib = row_base + ic * icg
            pltpu.sync_copy(src2_h.at[pl.ds(ib, icg)], idx_src)
            pltpu.sync_copy(dst2_h.at[pl.ds(ib, icg)], idx_dst)
            pltpu.sync_copy(ef_h.at[pl.ds(ib, icg)], idx_e)

            def comp(g, off):
                for j in range(GROUP // L):
                    ev = idx_e[g, pl.ds(j * L, L)]
                    m = (ev == 6) | (ev == 14) | (ev == 30)
                    plsc.store_compressed(
                        csrc.at[pl.ds(off, L)],
                        idx_src[g, pl.ds(j * L, L)], mask=m)
                    plsc.store_compressed(
                        cdst.at[pl.ds(off, L)],
                        idx_dst[g, pl.ds(j * L, L)], mask=m)
                    off = off + jnp.sum(m.astype(jnp.int32))
                return off
            off = lax.fori_loop(0, icg, comp, jnp.int32(0))
            run_compacted(off)
            return carry
        lax.fori_loop(0, my_gpt // icg, ichunk2, 0)

        plsc.subcore_barrier()
        @pl.when(s < NS - 1)
        def _():
            pltpu.sync_copy(acc.at[pl.ds(s * wrows, wrows)],
                            out_h.at[c, pl.ds(s * wrows, wrows)])

        @pl.when(s == NS - 1)
        def _():
            pltpu.sync_copy(acc.at[pl.ds((NS - 1) * wrows, wlast)],
                            out_h.at[c, pl.ds((NS - 1) * wrows, wlast)])

    return sc_kernel


def kernel(graph, trans_graph, graph_embedding, e_feat, weight):
    n_nodes, d = graph_embedding.shape
    e = graph.shape[1]

    bn = 1000
    emb = pl.pallas_call(
        _elu_body,
        out_shape=jax.ShapeDtypeStruct((n_nodes, d), jnp.float32),
        grid=(n_nodes // bn,),
        in_specs=[
            pl.BlockSpec((bn, d), lambda i: (i, 0)),
            pl.BlockSpec((1, d), lambda i: (0, 0)),
        ],
        out_specs=pl.BlockSpec((bn, d), lambda i: (i, 0)),
    )(graph_embedding, weight)

    gpt = -(-e // (NW * GROUP))
    gpt = -(-gpt // 16) * 16
    frac0 = 0.90
    gpt0 = int(round(2 * gpt * frac0 / 16)) * 16
    gpt1 = 2 * gpt - gpt0
    e_pad = NS * (gpt0 + gpt1) * GROUP
    pad = e_pad - e
    trash = n_nodes
    acc_rows = -(-(n_nodes + NS) // (NS * GROUP)) * (NS * GROUP)

    def prep(idx, padval):
        if pad:
            idx = jnp.concatenate(
                [idx, jnp.full((pad,), padval, jnp.int32)])
        return idx.reshape(-1, GROUP)

    src1 = prep(graph[0], 0)
    dst1 = prep(graph[1], trash)
    src2 = prep(trans_graph[0], 0)
    dst2 = prep(trans_graph[1], trash)
    ef = prep(e_feat, 5)

    sc_call = _make_sc_call(n_nodes, d, gpt0, gpt1, acc_rows, trash)
    partials = sc_call(emb, src1, dst1, src2, dst2, ef)

    out = pl.pallas_call(
        _add_body,
        out_shape=jax.ShapeDtypeStruct((n_nodes, d), jnp.float32),
        grid=(n_nodes // bn,),
        in_specs=[pl.BlockSpec((NC, bn, d), lambda i: (0, i, 0))],
        out_specs=pl.BlockSpec((bn, d), lambda i: (i, 0)),
    )(partials)
    return out


# speedup vs baseline: 1.4467x; 1.4467x over previous
import functools

import jax
import jax.numpy as jnp
from jax import lax
from jax.experimental import pallas as pl
from jax.experimental.pallas import tpu as pltpu
from jax.experimental.pallas import tpu_sc as plsc

L = 16
NC = 2
NS = 16
NW = NC * NS
GROUP = 128


def _elu_body(x_ref, w_ref, o_ref):
    x = x_ref[...] * w_ref[...]
    o_ref[...] = jnp.where(x > 0, x, jnp.exp(jnp.minimum(x, 0.0)) - 1.0)


def _add_body(p_ref, o_ref):
    o_ref[...] = p_ref[0] + p_ref[1]


def _make_sc_call(n_nodes, d, gpt0, gpt1, acc_rows, trash):
    icg = 16
    cbuf_len = (icg + 1) * GROUP + L
    wrows = -(-n_nodes // (NS * 8)) * 8
    wlast = n_nodes - (NS - 1) * wrows
    zper = acc_rows // NS
    mesh = plsc.VectorSubcoreMesh(core_axis_name="c", subcore_axis_name="s")

    @functools.partial(
        pl.kernel,
        out_type=jax.ShapeDtypeStruct((NC, n_nodes, d), jnp.float32),
        mesh=mesh,
        compiler_params=pltpu.CompilerParams(needs_layout_passes=False),
        scratch_types=[
            pltpu.VMEM((icg, GROUP), jnp.int32),
            pltpu.VMEM((icg, GROUP), jnp.int32),
            pltpu.VMEM((icg, GROUP), jnp.int32),
            pltpu.VMEM((cbuf_len,), jnp.int32),
            pltpu.VMEM((cbuf_len,), jnp.int32),
            pltpu.VMEM((2, GROUP), jnp.int32),
            pltpu.VMEM((GROUP,), jnp.float32),
            pltpu.VMEM((GROUP, d), jnp.float32),
            pltpu.VMEM((GROUP, d), jnp.float32),
            pltpu.VMEM_SHARED((acc_rows, d), jnp.float32),
            pltpu.SemaphoreType.DMA,
            pltpu.SemaphoreType.DMA,
            pltpu.SemaphoreType.DMA,
            pltpu.SemaphoreType.DMA,
        ],
    )
    def sc_kernel(emb_h, src1_h, dst1_h, src2_h, dst2_h, ef_h, out_h,
                  idx_src, idx_dst, idx_e, csrc, cdst, hrow, scale_v,
                  rows0, rows1, acc, gsem0, gsem1, ssem0, ssem1):
        rows_bufs = (rows0, rows1)
        gsems = (gsem0, gsem1)
        ssems = (ssem0, ssem1)
        c = lax.axis_index("c")
        s = lax.axis_index("s")

        def zrow(r, carry):
            for k in range(d // L):
                rows0[r, pl.ds(k * L, L)] = jnp.zeros((L,), jnp.float32)
            return carry
        lax.fori_loop(0, GROUP, zrow, 0)
        zbase = s * zper
        def zcp(i, carry):
            pltpu.sync_copy(rows0, acc.at[pl.ds(zbase + i * GROUP, GROUP)])
            return carry
        lax.fori_loop(0, zper // GROUP, zcp, 0)
        plsc.subcore_barrier()

        my_gpt = jnp.where(c == 0, gpt0, gpt1)
        row_base = jnp.where(c == 0, s * gpt0, NS * gpt0 + s * gpt1)

        tvec = jnp.full((L,), trash, jnp.int32) + s

        def run_compacted(off):
            for k in range(GROUP // L):
                csrc[pl.ds(off + k * L, L)] = jnp.zeros((L,), jnp.int32)
                cdst[pl.ds(off + k * L, L)] = tvec
            ng = off // GROUP + 1

            pltpu.async_copy(emb_h.at[csrc.at[pl.ds(0, GROUP)]], rows0,
                             gsem0)

            def pair2(gg, carry2):
                for b in range(2):
                    g2 = gg * 2 + b

                    @pl.when(g2 < ng)
                    def _():
                        rb, rnb = rows_bufs[b], rows_bufs[1 - b]

                        @pl.when(g2 + 1 < ng)
                        def _():
                            @pl.when(g2 >= 1)
                            def _():
                                pltpu.make_async_copy(
                                    rnb, acc.at[hrow.at[1 - b]],
                                    ssems[1 - b]).wait()
                            pltpu.async_copy(
                                emb_h.at[
                                    csrc.at[pl.ds((g2 + 1) * GROUP, GROUP)]],
                                rnb, gsems[1 - b])

                        pltpu.make_async_copy(
                            emb_h.at[csrc.at[pl.ds(0, GROUP)]], rb,
                            gsems[b]).wait()

                        for k in range(GROUP // L):
                            hrow[b, pl.ds(k * L, L)] = cdst[
                                pl.ds(g2 * GROUP + k * L, L)]

                        pltpu.async_copy(rb, acc.at[hrow.at[b]], ssems[b],
                                         add=True)
                return carry2
            lax.fori_loop(0, (icg + 2) // 2, pair2, 0)

            nm = ng % 2

            @pl.when(ng >= 2)
            def _():
                @pl.when(nm == 0)
                def _():
                    pltpu.make_async_copy(
                        rows0, acc.at[hrow.at[0]], ssem0).wait()

                @pl.when(nm == 1)
                def _():
                    pltpu.make_async_copy(
                        rows1, acc.at[hrow.at[1]], ssem1).wait()

            @pl.when(nm == 1)
            def _():
                pltpu.make_async_copy(
                    rows0, acc.at[hrow.at[0]], ssem0).wait()

            @pl.when(nm == 0)
            def _():
                pltpu.make_async_copy(
                    rows1, acc.at[hrow.at[1]], ssem1).wait()

        def ichunk1(ic, carry):
            ib = row_base + ic * icg
            pltpu.sync_copy(src1_h.at[pl.ds(ib, icg)], idx_src)
            pltpu.sync_copy(dst1_h.at[pl.ds(ib, icg)], idx_dst)
            pltpu.sync_copy(ef_h.at[pl.ds(ib, icg)], idx_e)

            pltpu.async_copy(emb_h.at[idx_src.at[0]], rows0, gsem0)

            def pair(gg, carry2):
                for b in range(2):
                    g = gg * 2 + b
                    rb, rnb = rows_bufs[b], rows_bufs[1 - b]

                    @pl.when(g + 1 < icg)
                    def _():
                        @pl.when(g >= 1)
                        def _():
                            pltpu.make_async_copy(
                                rnb, acc.at[idx_dst.at[0]],
                                ssems[1 - b]).wait()
                        pltpu.async_copy(
                            emb_h.at[idx_src.at[g + 1]], rnb, gsems[1 - b])

                    pltpu.make_async_copy(
                        emb_h.at[idx_src.at[g]], rb, gsems[b]).wait()

                    for j in range(GROUP // L):
                        ev = idx_e[g, pl.ds(j * L, L)]
                        m = (ev >= 0) & (ev <= 4)
                        scale_v[pl.ds(j * L, L)] = jnp.where(
                            m, jnp.float32(2.0), jnp.float32(1.0))

                    def mule(t, carry3):
                        sp = plsc.load_gather(
                            scale_v, [jnp.full((L,), t, jnp.int32)])
                        for k in range(d // L):
                            rb[t, pl.ds(k * L, L)] = (
                                rb[t, pl.ds(k * L, L)] * sp)
                        return carry3
                    lax.fori_loop(0, GROUP, mule, 0)

                    pltpu.async_copy(rb, acc.at[idx_dst.at[g]], ssems[b],
                                     add=True)
                return carry2
            lax.fori_loop(0, icg // 2, pair, 0)

            pltpu.make_async_copy(rows0, acc.at[idx_dst.at[0]],
                                  ssem0).wait()
            pltpu.make_async_copy(rows1, acc.at[idx_dst.at[0]],
                                  ssem1).wait()
            return carry
        lax.fori_loop(0, my_gpt // icg, ichunk1, 0)

        def ichunk2(ic, carry):
            ib = row_base + ic * icg
            pltpu.sync_copy(src2_h.at[pl.ds(ib, icg)], idx_src)
            pltpu.sync_copy(dst2_h.at[pl.ds(ib, icg)], idx_dst)
            pltpu.sync_copy(ef_h.at[pl.ds(ib, icg)], idx_e)

            def comp(g, off):
                for j in range(GROUP // L):
                    ev = idx_e[g, pl.ds(j * L, L)]
                    m = (ev == 6) | (ev == 14) | (ev == 30)
                    plsc.store_compressed(
                        csrc.at[pl.ds(off, L)],
                        idx_src[g, pl.ds(j * L, L)], mask=m)
                    plsc.store_compressed(
                        cdst.at[pl.ds(off, L)],
                        idx_dst[g, pl.ds(j * L, L)], mask=m)
                    off = off + jnp.sum(m.astype(jnp.int32))
                return off
            off = lax.fori_loop(0, icg, comp, jnp.int32(0))
            run_compacted(off)
            return carry
        lax.fori_loop(0, my_gpt // icg, ichunk2, 0)

        plsc.subcore_barrier()
        @pl.when(s < NS - 1)
        def _():
            pltpu.sync_copy(acc.at[pl.ds(s * wrows, wrows)],
                            out_h.at[c, pl.ds(s * wrows, wrows)])

        @pl.when(s == NS - 1)
        def _():
            pltpu.sync_copy(acc.at[pl.ds((NS - 1) * wrows, wlast)],
                            out_h.at[c, pl.ds((NS - 1) * wrows, wlast)])

    return sc_kernel


def kernel(graph, trans_graph, graph_embedding, e_feat, weight):
    n_nodes, d = graph_embedding.shape
    e = graph.shape[1]

    bn = 1000
    emb = pl.pallas_call(
        _elu_body,
        out_shape=jax.ShapeDtypeStruct((n_nodes, d), jnp.float32),
        grid=(n_nodes // bn,),
        in_specs=[
            pl.BlockSpec((bn, d), lambda i: (i, 0)),
            pl.BlockSpec((1, d), lambda i: (0, 0)),
        ],
        out_specs=pl.BlockSpec((bn, d), lambda i: (i, 0)),
    )(graph_embedding, weight)

    gpt = -(-e // (NW * GROUP))
    gpt = -(-gpt // 16) * 16
    frac0 = 0.90
    gpt0 = int(round(2 * gpt * frac0 / 16)) * 16
    gpt1 = 2 * gpt - gpt0
    e_pad = NS * (gpt0 + gpt1) * GROUP
    pad = e_pad - e
    trash = n_nodes
    acc_rows = -(-(n_nodes + NS) // (NS * GROUP)) * (NS * GROUP)

    def prep(idx, padval):
        if pad:
            idx = jnp.concatenate(
                [idx, jnp.full((pad,), padval, jnp.int32)])
        return idx.reshape(-1, GROUP)

    src1 = prep(graph[0], 0)
    dst1 = prep(graph[1], trash)
    src2 = prep(trans_graph[0], 0)
    dst2 = prep(trans_graph[1], trash)
    ef = prep(e_feat, 5)

    sc_call = _make_sc_call(n_nodes, d, gpt0, gpt1, acc_rows, trash)
    partials = sc_call(emb, src1, dst1, src2, dst2, ef)

    out = pl.pallas_call(
        _add_body,
        out_shape=jax.ShapeDtypeStruct((n_nodes, d), jnp.float32),
        grid=(n_nodes // bn,),
        in_specs=[pl.BlockSpec((NC, bn, d), lambda i: (0, i, 0))],
        out_specs=pl.BlockSpec((bn, d), lambda i: (i, 0)),
    )(partials)
    return out


# speedup vs baseline: 1.4498x; 1.0021x over previous
import functools

import jax
import jax.numpy as jnp
from jax import lax
from jax.experimental import pallas as pl
from jax.experimental.pallas import tpu as pltpu
from jax.experimental.pallas import tpu_sc as plsc

L = 16
NC = 2
NS = 16
NW = NC * NS
GROUP = 128


def _elu_body(x_ref, w_ref, o_ref):
    x = x_ref[...] * w_ref[...]
    o_ref[...] = jnp.where(x > 0, x, jnp.exp(jnp.minimum(x, 0.0)) - 1.0)


def _add_body(p_ref, o_ref):
    o_ref[...] = p_ref[0] + p_ref[1]


def _make_sc_call(n_nodes, d, gpt0, gpt1, acc_rows, trash):
    icg = 16
    cbuf_len = (icg + 1) * GROUP + L
    wrows = -(-n_nodes // (NS * 8)) * 8
    wlast = n_nodes - (NS - 1) * wrows
    zper = acc_rows // NS
    mesh = plsc.VectorSubcoreMesh(core_axis_name="c", subcore_axis_name="s")

    @functools.partial(
        pl.kernel,
        out_type=jax.ShapeDtypeStruct((NC, n_nodes, d), jnp.float32),
        mesh=mesh,
        compiler_params=pltpu.CompilerParams(needs_layout_passes=False),
        scratch_types=[
            pltpu.VMEM((icg, GROUP), jnp.int32),
            pltpu.VMEM((icg, GROUP), jnp.int32),
            pltpu.VMEM((icg, GROUP), jnp.int32),
            pltpu.VMEM((cbuf_len,), jnp.int32),
            pltpu.VMEM((cbuf_len,), jnp.int32),
            pltpu.VMEM((2, GROUP), jnp.int32),
            pltpu.VMEM((GROUP,), jnp.float32),
            pltpu.VMEM((GROUP, d), jnp.float32),
            pltpu.VMEM((GROUP, d), jnp.float32),
            pltpu.VMEM_SHARED((acc_rows, d), jnp.float32),
            pltpu.SemaphoreType.DMA,
            pltpu.SemaphoreType.DMA,
            pltpu.SemaphoreType.DMA,
            pltpu.SemaphoreType.DMA,
        ],
    )
    def sc_kernel(emb_h, src1_h, dst1_h, src2_h, dst2_h, ef_h, out_h,
                  idx_src, idx_dst, idx_e, csrc, cdst, hrow, scale_v,
                  rows0, rows1, acc, gsem0, gsem1, ssem0, ssem1):
        rows_bufs = (rows0, rows1)
        gsems = (gsem0, gsem1)
        ssems = (ssem0, ssem1)
        c = lax.axis_index("c")
        s = lax.axis_index("s")

        def zrow(r, carry):
            for k in range(d // L):
                rows0[r, pl.ds(k * L, L)] = jnp.zeros((L,), jnp.float32)
            return carry
        lax.fori_loop(0, GROUP, zrow, 0)
        zbase = s * zper
        def zcp(i, carry):
            pltpu.sync_copy(rows0, acc.at[pl.ds(zbase + i * GROUP, GROUP)])
            return carry
        lax.fori_loop(0, zper // GROUP, zcp, 0)
        plsc.subcore_barrier()

        my_gpt = jnp.where(c == 0, gpt0, gpt1)
        row_base = jnp.where(c == 0, s * gpt0, NS * gpt0 + s * gpt1)

        tvec = jnp.full((L,), trash, jnp.int32) + s

        def run_compacted(off):
            for k in range(GROUP // L):
                csrc[pl.ds(off + k * L, L)] = jnp.zeros((L,), jnp.int32)
                cdst[pl.ds(off + k * L, L)] = tvec
            ng = off // GROUP + 1

            pltpu.async_copy(emb_h.at[csrc.at[pl.ds(0, GROUP)]], rows0,
                             gsem0)

            def pair2(gg, carry2):
                for b in range(2):
                    g2 = gg * 2 + b

                    @pl.when(g2 < ng)
                    def _():
                        rb, rnb = rows_bufs[b], rows_bufs[1 - b]

                        @pl.when(g2 + 1 < ng)
                        def _():
                            @pl.when(g2 >= 1)
                            def _():
                                pltpu.make_async_copy(
                                    rnb, acc.at[hrow.at[1 - b]],
                                    ssems[1 - b]).wait()
                            pltpu.async_copy(
                                emb_h.at[
                                    csrc.at[pl.ds((g2 + 1) * GROUP, GROUP)]],
                                rnb, gsems[1 - b])

                        pltpu.make_async_copy(
                            emb_h.at[csrc.at[pl.ds(0, GROUP)]], rb,
                            gsems[b]).wait()

                        for k in range(GROUP // L):
                            hrow[b, pl.ds(k * L, L)] = cdst[
                                pl.ds(g2 * GROUP + k * L, L)]

                        pltpu.async_copy(rb, acc.at[hrow.at[b]], ssems[b],
                                         add=True)
                return carry2
            lax.fori_loop(0, (icg + 2) // 2, pair2, 0)

            nm = ng % 2

            @pl.when(ng >= 2)
            def _():
                @pl.when(nm == 0)
                def _():
                    pltpu.make_async_copy(
                        rows0, acc.at[hrow.at[0]], ssem0).wait()

                @pl.when(nm == 1)
                def _():
                    pltpu.make_async_copy(
                        rows1, acc.at[hrow.at[1]], ssem1).wait()

            @pl.when(nm == 1)
            def _():
                pltpu.make_async_copy(
                    rows0, acc.at[hrow.at[0]], ssem0).wait()

            @pl.when(nm == 0)
            def _():
                pltpu.make_async_copy(
                    rows1, acc.at[hrow.at[1]], ssem1).wait()

        def ichunk1(ic, carry):
            ib = row_base + ic * icg
            pltpu.async_copy(src1_h.at[pl.ds(ib, icg)], idx_src, gsem0)
            pltpu.async_copy(dst1_h.at[pl.ds(ib, icg)], idx_dst, gsem1)
            pltpu.async_copy(ef_h.at[pl.ds(ib, icg)], idx_e, ssem0)
            pltpu.make_async_copy(src1_h.at[pl.ds(ib, icg)], idx_src,
                                  gsem0).wait()
            pltpu.make_async_copy(dst1_h.at[pl.ds(ib, icg)], idx_dst,
                                  gsem1).wait()
            pltpu.make_async_copy(ef_h.at[pl.ds(ib, icg)], idx_e,
                                  ssem0).wait()

            pltpu.async_copy(emb_h.at[idx_src.at[0]], rows0, gsem0)

            def pair(gg, carry2):
                for b in range(2):
                    g = gg * 2 + b
                    rb, rnb = rows_bufs[b], rows_bufs[1 - b]

                    @pl.when(g + 1 < icg)
                    def _():
                        @pl.when(g >= 1)
                        def _():
                            pltpu.make_async_copy(
                                rnb, acc.at[idx_dst.at[0]],
                                ssems[1 - b]).wait()
                        pltpu.async_copy(
                            emb_h.at[idx_src.at[g + 1]], rnb, gsems[1 - b])

                    pltpu.make_async_copy(
                        emb_h.at[idx_src.at[g]], rb, gsems[b]).wait()

                    for j in range(GROUP // L):
                        ev = idx_e[g, pl.ds(j * L, L)]
                        m = (ev >= 0) & (ev <= 4)
                        scale_v[pl.ds(j * L, L)] = jnp.where(
                            m, jnp.float32(2.0), jnp.float32(1.0))

                    def mule(t, carry3):
                        sp = plsc.load_gather(
                            scale_v, [jnp.full((L,), t, jnp.int32)])
                        for k in range(d // L):
                            rb[t, pl.ds(k * L, L)] = (
                                rb[t, pl.ds(k * L, L)] * sp)
                        return carry3
                    lax.fori_loop(0, GROUP, mule, 0)

                    pltpu.async_copy(rb, acc.at[idx_dst.at[g]], ssems[b],
                                     add=True)
                return carry2
            lax.fori_loop(0, icg // 2, pair, 0)

            pltpu.make_async_copy(rows0, acc.at[idx_dst.at[0]],
                                  ssem0).wait()
            pltpu.make_async_copy(rows1, acc.at[idx_dst.at[0]],
                                  ssem1).wait()
            return carry
        lax.fori_loop(0, my_gpt // icg, ichunk1, 0)

        def ichunk2(ic, carry):
            ib = row_base + ic * icg
            pltpu.async_copy(src2_h.at[pl.ds(ib, icg)], idx_src, gsem0)
            pltpu.async_copy(dst2_h.at[pl.ds(ib, icg)], idx_dst, gsem1)
            pltpu.async_copy(ef_h.at[pl.ds(ib, icg)], idx_e, ssem0)
            pltpu.make_async_copy(src2_h.at[pl.ds(ib, icg)], idx_src,
                                  gsem0).wait()
            pltpu.make_async_copy(dst2_h.at[pl.ds(ib, icg)], idx_dst,
                                  gsem1).wait()
            pltpu.make_async_copy(ef_h.at[pl.ds(ib, icg)], idx_e,
                                  ssem0).wait()

            def comp(g, off):
                for j in range(GROUP // L):
                    ev = idx_e[g, pl.ds(j * L, L)]
                    m = (ev == 6) | (ev == 14) | (ev == 30)
                    plsc.store_compressed(
                        csrc.at[pl.ds(off, L)],
                        idx_src[g, pl.ds(j * L, L)], mask=m)
                    plsc.store_compressed(
                        cdst.at[pl.ds(off, L)],
                        idx_dst[g, pl.ds(j * L, L)], mask=m)
                    off = off + jnp.sum(m.astype(jnp.int32))
                return off
            off = lax.fori_loop(0, icg, comp, jnp.int32(0))
            run_compacted(off)
            return carry
        lax.fori_loop(0, my_gpt // icg, ichunk2, 0)

        plsc.subcore_barrier()
        @pl.when(s < NS - 1)
        def _():
            pltpu.sync_copy(acc.at[pl.ds(s * wrows, wrows)],
                            out_h.at[c, pl.ds(s * wrows, wrows)])

        @pl.when(s == NS - 1)
        def _():
            pltpu.sync_copy(acc.at[pl.ds((NS - 1) * wrows, wlast)],
                            out_h.at[c, pl.ds((NS - 1) * wrows, wlast)])

    return sc_kernel


def kernel(graph, trans_graph, graph_embedding, e_feat, weight):
    n_nodes, d = graph_embedding.shape
    e = graph.shape[1]

    bn = 1000
    emb = pl.pallas_call(
        _elu_body,
        out_shape=jax.ShapeDtypeStruct((n_nodes, d), jnp.float32),
        grid=(n_nodes // bn,),
        in_specs=[
            pl.BlockSpec((bn, d), lambda i: (i, 0)),
            pl.BlockSpec((1, d), lambda i: (0, 0)),
        ],
        out_specs=pl.BlockSpec((bn, d), lambda i: (i, 0)),
    )(graph_embedding, weight)

    gpt = -(-e // (NW * GROUP))
    gpt = -(-gpt // 16) * 16
    frac0 = 0.90
    gpt0 = int(round(2 * gpt * frac0 / 16)) * 16
    gpt1 = 2 * gpt - gpt0
    e_pad = NS * (gpt0 + gpt1) * GROUP
    pad = e_pad - e
    trash = n_nodes
    acc_rows = -(-(n_nodes + NS) // (NS * GROUP)) * (NS * GROUP)

    def prep(idx, padval):
        if pad:
            idx = jnp.concatenate(
                [idx, jnp.full((pad,), padval, jnp.int32)])
        return idx.reshape(-1, GROUP)

    src1 = prep(graph[0], 0)
    dst1 = prep(graph[1], trash)
    src2 = prep(trans_graph[0], 0)
    dst2 = prep(trans_graph[1], trash)
    ef = prep(e_feat, 5)

    sc_call = _make_sc_call(n_nodes, d, gpt0, gpt1, acc_rows, trash)
    partials = sc_call(emb, src1, dst1, src2, dst2, ef)

    out = pl.pallas_call(
        _add_body,
        out_shape=jax.ShapeDtypeStruct((n_nodes, d), jnp.float32),
        grid=(n_nodes // bn,),
        in_specs=[pl.BlockSpec((NC, bn, d), lambda i: (0, i, 0))],
        out_specs=pl.BlockSpec((bn, d), lambda i: (i, 0)),
    )(partials)
    return out
